# baseline (speedup 1.0000x reference)
import functools

import jax
import jax.numpy as jnp
from jax import lax
from jax.experimental import pallas as pl
from jax.experimental.pallas import tpu as pltpu
from jax.experimental.pallas import tpu_sc as plsc

N_NODES = 10000
N_EDGES = 320000
F = 128

NC = 2
NS = 16
NW = NC * NS
CH = 80
EW_TOT = N_EDGES // NW
CH_TOT = EW_TOT // CH
PARTS = ((0, 62), (62, 63))
N_PAD = 10240
ROWS_PER_TILE = N_PAD // NS
ZR = 128

_mesh = plsc.VectorSubcoreMesh(core_axis_name="c", subcore_axis_name="s")
_sc_params = pltpu.CompilerParams(needs_layout_passes=False,
                                  use_tc_tiling_on_sc=False)


def _pre_body(h_ref, wa_ref, wb_ref, a_ref, b_ref):
    hb = h_ref[...]
    a_ref[...] = jnp.dot(
        hb, wa_ref[...], preferred_element_type=jnp.float32).astype(jnp.bfloat16)
    b_ref[...] = jnp.dot(
        hb, wb_ref[...], preferred_element_type=jnp.float32).astype(jnp.bfloat16)


def _precompute(h, wa, wb):
    BN = 2000
    return pl.pallas_call(
        _pre_body,
        grid=(N_NODES // BN,),
        in_specs=[
            pl.BlockSpec((BN, F), lambda i: (i, 0)),
            pl.BlockSpec((F, F), lambda i: (0, 0)),
            pl.BlockSpec((F, F), lambda i: (0, 0)),
        ],
        out_specs=[
            pl.BlockSpec((BN, F), lambda i: (i, 0)),
            pl.BlockSpec((BN, F), lambda i: (i, 0)),
        ],
        out_shape=[jax.ShapeDtypeStruct((N_NODES, F), jnp.bfloat16)] * 2,
    )(h, wa, wb)


def _make_gather(c0, nch):
    epw = nch * CH
    e_part = NW * epw

    @functools.partial(
        pl.kernel,
        out_type=[
            jax.ShapeDtypeStruct((e_part, F // 2), jnp.int32),
            jax.ShapeDtypeStruct((e_part,), jnp.float32),
        ],
        mesh=_mesh,
        scratch_types=[
            pltpu.VMEM((epw,), jnp.int32),
            pltpu.VMEM((epw,), jnp.int32),
            pltpu.VMEM((3 * N_NODES,), jnp.float32),
            pltpu.VMEM((CH, F // 2), jnp.int32),
            pltpu.VMEM((CH, F // 2), jnp.int32),
            pltpu.VMEM((CH, F // 2), jnp.int32),
            pltpu.VMEM((CH, F // 2), jnp.int32),
            pltpu.VMEM((CH, F // 2), jnp.int32),
            pltpu.VMEM((CH, F // 2), jnp.int32),
            pltpu.VMEM((CH,), jnp.float32),
            pltpu.VMEM((CH,), jnp.float32),
            pltpu.SemaphoreType.DMA, pltpu.SemaphoreType.DMA,
            pltpu.SemaphoreType.DMA, pltpu.SemaphoreType.DMA,
            pltpu.SemaphoreType.DMA, pltpu.SemaphoreType.DMA,
            pltpu.SemaphoreType.DMA, pltpu.SemaphoreType.DMA,
        ],
        compiler_params=_sc_params,
    )
    def gather(send_hbm, recv_hbm, a_hbm, b_hbm, cf_hbm, m0_hbm, d2_hbm,
               sidx_all, ridx_all, cf_v,
               buf_a0, buf_a1, buf_b0, buf_b1, buf_w0, buf_w1, d2v0, d2v1,
               sem_a0, sem_a1, sem_b0, sem_b1, sem_w0, sem_w1, sem_d0, sem_d1):
        buf_a = (buf_a0, buf_a1)
        buf_b = (buf_b0, buf_b1)
        buf_w = (buf_w0, buf_w1)
        d2v = (d2v0, d2v1)
        sem_a = (sem_a0, sem_a1)
        sem_b = (sem_b0, sem_b1)
        sem_w = (sem_w0, sem_w1)
        sem_d = (sem_d0, sem_d1)

        wid = lax.axis_index("s") * NC + lax.axis_index("c")
        base_in = wid * EW_TOT + c0 * CH
        base_out = wid * epw
        pltpu.sync_copy(cf_hbm, cf_v)
        pltpu.sync_copy(send_hbm.at[pl.ds(base_in, epw)], sidx_all)
        pltpu.sync_copy(recv_hbm.at[pl.ds(base_in, epw)], ridx_all)

        def start_gathers(g, b):
            off = g * CH
            pltpu.async_copy(a_hbm.at[sidx_all.at[pl.ds(off, CH)]], buf_a[b], sem_a[b])
            pltpu.async_copy(b_hbm.at[ridx_all.at[pl.ds(off, CH)]], buf_b[b], sem_b[b])

        def finish(g, b):
            @pl.when(g >= 2)
            def _():
                pltpu.make_async_copy(buf_w[b], m0_hbm.at[pl.ds(base_out, CH)], sem_w[b]).wait()
                pltpu.make_async_copy(d2v[b], d2_hbm.at[pl.ds(base_out, CH)], sem_d[b]).wait()

            for i in range(CH // 16):
                sl = pl.ds(g * CH + i * 16, 16)
                s3 = sidx_all[sl] * 3
                r3 = ridx_all[sl] * 3
                dx = plsc.load_gather(cf_v, [s3]) - plsc.load_gather(cf_v, [r3])
                dy = plsc.load_gather(cf_v, [s3 + 1]) - plsc.load_gather(cf_v, [r3 + 1])
                dz = plsc.load_gather(cf_v, [s3 + 2]) - plsc.load_gather(cf_v, [r3 + 2])
                d2v[b][pl.ds(i * 16, 16)] = dx * dx + dy * dy + dz * dz

            pltpu.make_async_copy(a_hbm.at[sidx_all.at[pl.ds(0, CH)]], buf_a[b], sem_a[b]).wait()
            pltpu.make_async_copy(b_hbm.at[ridx_all.at[pl.ds(0, CH)]], buf_b[b], sem_b[b]).wait()

            def add_body(r2, c):
                for rr in range(2):
                    r = r2 * 2 + rr
                    for col in range(F // 32):
                        sl2 = pl.ds(col * 16, 16)
                        va = plsc.bitcast(buf_a[b][r, sl2], jnp.bfloat16)
                        vb = plsc.bitcast(buf_b[b][r, sl2], jnp.bfloat16)
                        buf_w[b][r, sl2] = plsc.bitcast(va + vb, jnp.int32)
                return c

            lax.fori_loop(0, CH // 2, add_body, 0)
            pltpu.async_copy(buf_w[b], m0_hbm.at[pl.ds(base_out + g * CH, CH)], sem_w[b])
            pltpu.async_copy(d2v[b], d2_hbm.at[pl.ds(base_out + g * CH, CH)], sem_d[b])

            @pl.when(g < nch - 2)
            def _():
                start_gathers(g + 2, b)

        start_gathers(0, 0)
        start_gathers(1, 1)

        def pair(i2, c):
            finish(2 * i2, 0)
            finish(2 * i2 + 1, 1)
            return c

        if nch % 2:
            lax.fori_loop(0, (nch - 1) // 2, pair, 0)
            finish(nch - 1, 0)
        else:
            lax.fori_loop(0, (nch - 2) // 2, pair, 0)
            finish(nch - 2, 0)
            finish(nch - 1, 1)
        for b in range(2):
            pltpu.make_async_copy(buf_w[b], m0_hbm.at[pl.ds(base_out, CH)], sem_w[b]).wait()
            pltpu.make_async_copy(d2v[b], d2_hbm.at[pl.ds(base_out, CH)], sem_d[b]).wait()

    return gather


def _edge_mlp_body(m0_ref, d2_ref, w1d_ref, b1_ref, w2_ref, b2_ref, out_ref):
    x = (m0_ref[...].astype(jnp.float32)
         + jnp.sqrt(d2_ref[...]) * w1d_ref[...] + b1_ref[...])
    x = x * jax.nn.sigmoid(x)
    y = jnp.dot(x, w2_ref[...], preferred_element_type=jnp.float32) + b2_ref[...]
    out_ref[...] = y * jax.nn.sigmoid(y)


def _edge_mlp(m0, d2, w1d, b1, w2, b2):
    e_part = m0.shape[0]
    BE = 1280
    return pl.pallas_call(
        _edge_mlp_body,
        grid=(e_part // BE,),
        in_specs=[
            pl.BlockSpec((BE, F), lambda i: (i, 0)),
            pl.BlockSpec((BE, 1), lambda i: (i, 0)),
            pl.BlockSpec((1, F), lambda i: (0, 0)),
            pl.BlockSpec((1, F), lambda i: (0, 0)),
            pl.BlockSpec((F, F), lambda i: (0, 0)),
            pl.BlockSpec((1, F), lambda i: (0, 0)),
        ],
        out_specs=pl.BlockSpec((BE, F), lambda i: (i, 0)),
        out_shape=jax.ShapeDtypeStruct((e_part, F), jnp.float32),
    )(m0, d2, w1d, b1, w2, b2)


def _make_scatter(c0, nch):
    epw = nch * CH

    @functools.partial(
        pl.kernel,
        out_type=jax.ShapeDtypeStruct((NC, N_PAD, F), jnp.float32),
        mesh=_mesh,
        scratch_types=[
            pltpu.VMEM((CH,), jnp.int32),
            pltpu.VMEM((CH,), jnp.int32),
            pltpu.VMEM((CH, F), jnp.float32),
            pltpu.VMEM((CH, F), jnp.float32),
            pltpu.VMEM((ZR, F), jnp.float32),
            pltpu.VMEM_SHARED((N_PAD, F), jnp.float32),
            pltpu.SemaphoreType.DMA, pltpu.SemaphoreType.DMA,
            pltpu.SemaphoreType.DMA, pltpu.SemaphoreType.DMA,
        ],
        compiler_params=_sc_params,
    )
    def scatter(recv_hbm, msg_hbm, out_hbm, ridx0, ridx1, msg0, msg1, z_v,
                agg_sh, sem_i0, sem_i1, sem_m0, sem_m1):
        ridx = (ridx0, ridx1)
        msg_v = (msg0, msg1)
        sem_i = (sem_i0, sem_i1)
        sem_m = (sem_m0, sem_m1)
        cid = lax.axis_index("c")
        sid = lax.axis_index("s")
        wid = sid * NC + cid
        base_in = wid * EW_TOT + c0 * CH
        base_msg = wid * epw

        def start_loads(g, b):
            pltpu.async_copy(recv_hbm.at[pl.ds(base_in + g * CH, CH)], ridx[b], sem_i[b])
            pltpu.async_copy(msg_hbm.at[pl.ds(base_msg + g * CH, CH)], msg_v[b], sem_m[b])

        start_loads(0, 0)
        start_loads(1, 1)

        def zero_body(i, c):
            for col in range(F // 16):
                z_v[i, pl.ds(col * 16, 16)] = jnp.zeros((16,), jnp.float32)
            return c

        lax.fori_loop(0, ZR, zero_body, 0)
        for j in range(ROWS_PER_TILE // ZR):
            pltpu.sync_copy(z_v, agg_sh.at[pl.ds(sid * ROWS_PER_TILE + j * ZR, ZR)])
        plsc.subcore_barrier()

        def finish(g, b):
            pltpu.make_async_copy(recv_hbm.at[pl.ds(base_in, CH)], ridx[b], sem_i[b]).wait()
            pltpu.make_async_copy(msg_hbm.at[pl.ds(base_msg, CH)], msg_v[b], sem_m[b]).wait()
            pltpu.sync_copy(msg_v[b], agg_sh.at[ridx[b]], add=True)

            @pl.when(g < nch - 2)
            def _():
                start_loads(g + 2, b)

        def pair(i2, c):
            finish(2 * i2, 0)
            finish(2 * i2 + 1, 1)
            return c

        if nch % 2:
            lax.fori_loop(0, (nch - 1) // 2, pair, 0)
            finish(nch - 1, 0)
        else:
            lax.fori_loop(0, (nch - 2) // 2, pair, 0)
            finish(nch - 2, 0)
            finish(nch - 1, 1)

        plsc.subcore_barrier()
        pltpu.sync_copy(
            agg_sh.at[pl.ds(sid * ROWS_PER_TILE, ROWS_PER_TILE)],
            out_hbm.at[cid].at[pl.ds(sid * ROWS_PER_TILE, ROWS_PER_TILE)],
        )

    return scatter


_gathers = tuple(_make_gather(c0, n) for c0, n in PARTS)
_scatters = tuple(_make_scatter(c0, n) for c0, n in PARTS)


def _update_body(h_ref, g0_ref, g1_ref, g2_ref, g3_ref, ua_ref, ub_ref,
                 ub1_ref, u2_ref, ub2_ref, out_ref):
    hb = h_ref[...]
    agg = (g0_ref[...] + g1_ref[...]) + (g2_ref[...] + g3_ref[...])
    u = (jnp.dot(hb, ua_ref[...], preferred_element_type=jnp.float32)
         + jnp.dot(agg, ub_ref[...], preferred_element_type=jnp.float32)
         + ub1_ref[...])
    u = u * jax.nn.sigmoid(u)
    out_ref[...] = hb + jnp.dot(u, u2_ref[...],
                                preferred_element_type=jnp.float32) + ub2_ref[...]


def _update(h, gs, ua, ub, ub1, u2, ub2):
    BN = 2000
    blk = pl.BlockSpec((BN, F), lambda i: (i, 0))
    full = pl.BlockSpec((F, F), lambda i: (0, 0))
    row = pl.BlockSpec((1, F), lambda i: (0, 0))
    return pl.pallas_call(
        _update_body,
        grid=(N_NODES // BN,),
        in_specs=[blk, blk, blk, blk, blk, full, full, row, full, row],
        out_specs=blk,
        out_shape=jax.ShapeDtypeStruct((N_NODES, F), jnp.float32),
    )(h, *gs, ua, ub, ub1, u2, ub2)


def kernel(h, coords, edge_index, W1, b1, W2, b2, U1, ub1, U2, ub2):
    sender = edge_index[0].astype(jnp.int32)
    receiver = edge_index[1].astype(jnp.int32)
    cflat = coords.reshape(-1)

    a, b = _precompute(h, W1[:F], W1[F:2 * F])
    a32 = jax.lax.bitcast_convert_type(a.reshape(N_NODES, F // 2, 2), jnp.int32)
    b32 = jax.lax.bitcast_convert_type(b.reshape(N_NODES, F // 2, 2), jnp.int32)
    w1d = W1[2 * F].reshape(1, F)
    b1r = b1.reshape(1, F)
    b2r = b2.reshape(1, F)

    partials = []
    for p in range(len(PARTS)):
        m0_32, d2 = _gathers[p](sender, receiver, a32, b32, cflat)
        m0 = jax.lax.bitcast_convert_type(m0_32, jnp.bfloat16).reshape(-1, F)
        msgs = _edge_mlp(m0, d2.reshape(-1, 1), w1d, b1r, W2, b2r)
        parts = _scatters[p](receiver, msgs)
        partials.extend([parts[0], parts[1]])

    return _update(h, partials, U1[:F], U1[F:], ub1.reshape(1, F),
                   U2, ub2.reshape(1, F))


# speedup vs baseline: 2.1748x; 2.1748x over previous
import functools

import jax
import jax.numpy as jnp
from jax import lax
from jax.experimental import pallas as pl
from jax.experimental.pallas import tpu as pltpu
from jax.experimental.pallas import tpu_sc as plsc

N_NODES = 10000
N_EDGES = 320000
F = 128

NC = 2
NS = 16
NW = NC * NS
CH = 80
EW_TOT = N_EDGES // NW
CH_TOT = EW_TOT // CH
PARTS = ((0, 62), (62, 63))
N_PAD = 10240
ROWS_PER_TILE = N_PAD // NS
ZR = 128

_mesh = plsc.VectorSubcoreMesh(core_axis_name="c", subcore_axis_name="s")
_sc_params = pltpu.CompilerParams(needs_layout_passes=False)


def _pre_body(h_ref, wa_ref, wb_ref, a_ref, b_ref):
    hb = h_ref[...]
    a_ref[...] = jnp.dot(hb, wa_ref[...], preferred_element_type=jnp.float32)
    b_ref[...] = jnp.dot(hb, wb_ref[...], preferred_element_type=jnp.float32)


def _precompute(h, wa, wb):
    BN = 2000
    return pl.pallas_call(
        _pre_body,
        grid=(N_NODES // BN,),
        in_specs=[
            pl.BlockSpec((BN, F), lambda i: (i, 0)),
            pl.BlockSpec((F, F), lambda i: (0, 0)),
            pl.BlockSpec((F, F), lambda i: (0, 0)),
        ],
        out_specs=[
            pl.BlockSpec((BN, F), lambda i: (i, 0)),
            pl.BlockSpec((BN, F), lambda i: (i, 0)),
        ],
        out_shape=[jax.ShapeDtypeStruct((N_NODES, F), jnp.float32)] * 2,
    )(h, wa, wb)


def _make_gather(c0, nch):
    epw = nch * CH
    e_part = NW * epw

    @functools.partial(
        pl.kernel,
        out_type=[
            jax.ShapeDtypeStruct((e_part, F), jnp.float32),
            jax.ShapeDtypeStruct((e_part,), jnp.float32),
        ],
        mesh=_mesh,
        scratch_types=[
            pltpu.VMEM((epw,), jnp.int32),
            pltpu.VMEM((epw,), jnp.int32),
            pltpu.VMEM((3 * N_NODES,), jnp.float32),
            pltpu.VMEM((CH, F), jnp.float32),
            pltpu.VMEM((CH, F), jnp.float32),
            pltpu.VMEM((CH, F), jnp.float32),
            pltpu.VMEM((CH, F), jnp.float32),
            pltpu.VMEM((CH, F), jnp.float32),
            pltpu.VMEM((CH, F), jnp.float32),
            pltpu.VMEM((CH,), jnp.float32),
            pltpu.VMEM((CH,), jnp.float32),
            pltpu.SemaphoreType.DMA, pltpu.SemaphoreType.DMA,
            pltpu.SemaphoreType.DMA, pltpu.SemaphoreType.DMA,
            pltpu.SemaphoreType.DMA, pltpu.SemaphoreType.DMA,
            pltpu.SemaphoreType.DMA, pltpu.SemaphoreType.DMA,
        ],
        compiler_params=_sc_params,
    )
    def gather(send_hbm, recv_hbm, a_hbm, b_hbm, cf_hbm, m0_hbm, d2_hbm,
               sidx_all, ridx_all, cf_v,
               buf_a0, buf_a1, buf_b0, buf_b1, buf_w0, buf_w1, d2v0, d2v1,
               sem_a0, sem_a1, sem_b0, sem_b1, sem_w0, sem_w1, sem_d0, sem_d1):
        buf_a = (buf_a0, buf_a1)
        buf_b = (buf_b0, buf_b1)
        buf_w = (buf_w0, buf_w1)
        d2v = (d2v0, d2v1)
        sem_a = (sem_a0, sem_a1)
        sem_b = (sem_b0, sem_b1)
        sem_w = (sem_w0, sem_w1)
        sem_d = (sem_d0, sem_d1)

        wid = lax.axis_index("s") * NC + lax.axis_index("c")
        base_in = wid * EW_TOT + c0 * CH
        base_out = wid * epw
        pltpu.sync_copy(cf_hbm, cf_v)
        pltpu.sync_copy(send_hbm.at[pl.ds(base_in, epw)], sidx_all)
        pltpu.sync_copy(recv_hbm.at[pl.ds(base_in, epw)], ridx_all)

        def start_gathers(g, b):
            off = g * CH
            pltpu.async_copy(a_hbm.at[sidx_all.at[pl.ds(off, CH)]], buf_a[b], sem_a[b])
            pltpu.async_copy(b_hbm.at[ridx_all.at[pl.ds(off, CH)]], buf_b[b], sem_b[b])

        def finish(g, b):
            @pl.when(g >= 2)
            def _():
                pltpu.make_async_copy(buf_w[b], m0_hbm.at[pl.ds(base_out, CH)], sem_w[b]).wait()
                pltpu.make_async_copy(d2v[b], d2_hbm.at[pl.ds(base_out, CH)], sem_d[b]).wait()

            for i in range(CH // 16):
                sl = pl.ds(g * CH + i * 16, 16)
                s3 = sidx_all[sl] * 3
                r3 = ridx_all[sl] * 3
                dx = plsc.load_gather(cf_v, [s3]) - plsc.load_gather(cf_v, [r3])
                dy = plsc.load_gather(cf_v, [s3 + 1]) - plsc.load_gather(cf_v, [r3 + 1])
                dz = plsc.load_gather(cf_v, [s3 + 2]) - plsc.load_gather(cf_v, [r3 + 2])
                d2v[b][pl.ds(i * 16, 16)] = dx * dx + dy * dy + dz * dz

            pltpu.make_async_copy(a_hbm.at[sidx_all.at[pl.ds(0, CH)]], buf_a[b], sem_a[b]).wait()
            pltpu.make_async_copy(b_hbm.at[ridx_all.at[pl.ds(0, CH)]], buf_b[b], sem_b[b]).wait()

            def add_body(r2, c):
                for rr in range(2):
                    r = r2 * 2 + rr
                    for col in range(F // 16):
                        sl2 = pl.ds(col * 16, 16)
                        buf_w[b][r, sl2] = buf_a[b][r, sl2] + buf_b[b][r, sl2]
                return c

            lax.fori_loop(0, CH // 2, add_body, 0)
            pltpu.async_copy(buf_w[b], m0_hbm.at[pl.ds(base_out + g * CH, CH)], sem_w[b])
            pltpu.async_copy(d2v[b], d2_hbm.at[pl.ds(base_out + g * CH, CH)], sem_d[b])

            @pl.when(g < nch - 2)
            def _():
                start_gathers(g + 2, b)

        start_gathers(0, 0)
        start_gathers(1, 1)

        def pair(i2, c):
            finish(2 * i2, 0)
            finish(2 * i2 + 1, 1)
            return c

        if nch % 2:
            lax.fori_loop(0, (nch - 1) // 2, pair, 0)
            finish(nch - 1, 0)
        else:
            lax.fori_loop(0, (nch - 2) // 2, pair, 0)
            finish(nch - 2, 0)
            finish(nch - 1, 1)
        for b in range(2):
            pltpu.make_async_copy(buf_w[b], m0_hbm.at[pl.ds(base_out, CH)], sem_w[b]).wait()
            pltpu.make_async_copy(d2v[b], d2_hbm.at[pl.ds(base_out, CH)], sem_d[b]).wait()

    return gather


def _edge_mlp_body(m0_ref, d2_ref, w1d_ref, b1_ref, w2_ref, b2_ref, out_ref):
    x = m0_ref[...] + jnp.sqrt(d2_ref[...]) * w1d_ref[...] + b1_ref[...]
    x = x * jax.nn.sigmoid(x)
    y = jnp.dot(x, w2_ref[...], preferred_element_type=jnp.float32) + b2_ref[...]
    out_ref[...] = y * jax.nn.sigmoid(y)


def _edge_mlp(m0, d2, w1d, b1, w2, b2):
    e_part = m0.shape[0]
    BE = 1280
    return pl.pallas_call(
        _edge_mlp_body,
        grid=(e_part // BE,),
        in_specs=[
            pl.BlockSpec((BE, F), lambda i: (i, 0)),
            pl.BlockSpec((BE, 1), lambda i: (i, 0)),
            pl.BlockSpec((1, F), lambda i: (0, 0)),
            pl.BlockSpec((1, F), lambda i: (0, 0)),
            pl.BlockSpec((F, F), lambda i: (0, 0)),
            pl.BlockSpec((1, F), lambda i: (0, 0)),
        ],
        out_specs=pl.BlockSpec((BE, F), lambda i: (i, 0)),
        out_shape=jax.ShapeDtypeStruct((e_part, F), jnp.float32),
    )(m0, d2, w1d, b1, w2, b2)


def _make_scatter(c0, nch):
    epw = nch * CH

    @functools.partial(
        pl.kernel,
        out_type=jax.ShapeDtypeStruct((NC, N_PAD, F), jnp.float32),
        mesh=_mesh,
        scratch_types=[
            pltpu.VMEM((CH,), jnp.int32),
            pltpu.VMEM((CH,), jnp.int32),
            pltpu.VMEM((CH, F), jnp.float32),
            pltpu.VMEM((CH, F), jnp.float32),
            pltpu.VMEM((ZR, F), jnp.float32),
            pltpu.VMEM_SHARED((N_PAD, F), jnp.float32),
            pltpu.SemaphoreType.DMA, pltpu.SemaphoreType.DMA,
            pltpu.SemaphoreType.DMA, pltpu.SemaphoreType.DMA,
        ],
        compiler_params=_sc_params,
    )
    def scatter(recv_hbm, msg_hbm, out_hbm, ridx0, ridx1, msg0, msg1, z_v,
                agg_sh, sem_i0, sem_i1, sem_m0, sem_m1):
        ridx = (ridx0, ridx1)
        msg_v = (msg0, msg1)
        sem_i = (sem_i0, sem_i1)
        sem_m = (sem_m0, sem_m1)
        cid = lax.axis_index("c")
        sid = lax.axis_index("s")
        wid = sid * NC + cid
        base_in = wid * EW_TOT + c0 * CH
        base_msg = wid * epw

        def start_loads(g, b):
            pltpu.async_copy(recv_hbm.at[pl.ds(base_in + g * CH, CH)], ridx[b], sem_i[b])
            pltpu.async_copy(msg_hbm.at[pl.ds(base_msg + g * CH, CH)], msg_v[b], sem_m[b])

        start_loads(0, 0)
        start_loads(1, 1)

        def zero_body(i, c):
            for col in range(F // 16):
                z_v[i, pl.ds(col * 16, 16)] = jnp.zeros((16,), jnp.float32)
            return c

        lax.fori_loop(0, ZR, zero_body, 0)
        for j in range(ROWS_PER_TILE // ZR):
            pltpu.sync_copy(z_v, agg_sh.at[pl.ds(sid * ROWS_PER_TILE + j * ZR, ZR)])
        plsc.subcore_barrier()

        def finish(g, b):
            pltpu.make_async_copy(recv_hbm.at[pl.ds(base_in, CH)], ridx[b], sem_i[b]).wait()
            pltpu.make_async_copy(msg_hbm.at[pl.ds(base_msg, CH)], msg_v[b], sem_m[b]).wait()
            pltpu.sync_copy(msg_v[b], agg_sh.at[ridx[b]], add=True)

            @pl.when(g < nch - 2)
            def _():
                start_loads(g + 2, b)

        def pair(i2, c):
            finish(2 * i2, 0)
            finish(2 * i2 + 1, 1)
            return c

        if nch % 2:
            lax.fori_loop(0, (nch - 1) // 2, pair, 0)
            finish(nch - 1, 0)
        else:
            lax.fori_loop(0, (nch - 2) // 2, pair, 0)
            finish(nch - 2, 0)
            finish(nch - 1, 1)

        plsc.subcore_barrier()
        pltpu.sync_copy(
            agg_sh.at[pl.ds(sid * ROWS_PER_TILE, ROWS_PER_TILE)],
            out_hbm.at[cid].at[pl.ds(sid * ROWS_PER_TILE, ROWS_PER_TILE)],
        )

    return scatter


_gathers = tuple(_make_gather(c0, n) for c0, n in PARTS)
_scatters = tuple(_make_scatter(c0, n) for c0, n in PARTS)


def _update_body(h_ref, g0_ref, g1_ref, g2_ref, g3_ref, ua_ref, ub_ref,
                 ub1_ref, u2_ref, ub2_ref, out_ref):
    hb = h_ref[...]
    agg = (g0_ref[...] + g1_ref[...]) + (g2_ref[...] + g3_ref[...])
    u = (jnp.dot(hb, ua_ref[...], preferred_element_type=jnp.float32)
         + jnp.dot(agg, ub_ref[...], preferred_element_type=jnp.float32)
         + ub1_ref[...])
    u = u * jax.nn.sigmoid(u)
    out_ref[...] = hb + jnp.dot(u, u2_ref[...],
                                preferred_element_type=jnp.float32) + ub2_ref[...]


def _update(h, gs, ua, ub, ub1, u2, ub2):
    BN = 2000
    blk = pl.BlockSpec((BN, F), lambda i: (i, 0))
    full = pl.BlockSpec((F, F), lambda i: (0, 0))
    row = pl.BlockSpec((1, F), lambda i: (0, 0))
    return pl.pallas_call(
        _update_body,
        grid=(N_NODES // BN,),
        in_specs=[blk, blk, blk, blk, blk, full, full, row, full, row],
        out_specs=blk,
        out_shape=jax.ShapeDtypeStruct((N_NODES, F), jnp.float32),
    )(h, *gs, ua, ub, ub1, u2, ub2)


def kernel(h, coords, edge_index, W1, b1, W2, b2, U1, ub1, U2, ub2):
    sender = edge_index[0].astype(jnp.int32)
    receiver = edge_index[1].astype(jnp.int32)
    cflat = coords.reshape(-1)

    a, b = _precompute(h, W1[:F], W1[F:2 * F])
    w1d = W1[2 * F].reshape(1, F)
    b1r = b1.reshape(1, F)
    b2r = b2.reshape(1, F)

    partials = []
    for p in range(len(PARTS)):
        m0, d2 = _gathers[p](sender, receiver, a, b, cflat)
        msgs = _edge_mlp(m0, d2.reshape(-1, 1), w1d, b1r, W2, b2r)
        parts = _scatters[p](receiver, msgs)
        partials.extend([parts[0], parts[1]])

    return _update(h, partials, U1[:F], U1[F:], ub1.reshape(1, F),
                   U2, ub2.reshape(1, F))
